# Initial kernel scaffold; baseline (speedup 1.0000x reference)
import jax
import jax.numpy as jnp
from jax.experimental import pallas as pl


def kernel(pts3D, src, default_feature):
    raise NotImplementedError("write your pallas kernel here")


# speedup vs baseline: 5.4157x; 5.4157x over previous
import functools

import jax
import jax.numpy as jnp
from jax import lax
from jax.experimental import pallas as pl
from jax.experimental.pallas import tpu as pltpu
from jax.experimental.pallas import tpu_sc as plsc

SIZE = 256
KSLOTS = 8
RADIUS_NDC = 2.0 * 1.5 / SIZE
R2 = RADIUS_NDC * RADIUS_NDC
C = 64

NC = 2
NS = 16
NW = NC * NS
L = 16

NPIX = 2 * SIZE * SIZE
PW = NPIX // NW
CP = 128
G = 16
NCHUNK = PW // G


def _compose_kernel(idx_hbm, d2_hbm, feats_hbm, out_hbm, idxv, d2v, rows,
                    outv, sem):
    wid = lax.axis_index("s") * NC + lax.axis_index("c")
    base = wid * PW
    lanes = lax.iota(jnp.int32, L)
    inv_r2 = jnp.float32(1.0 / R2)

    def chunk(ci, carry):
        p0 = base + ci * G
        pltpu.sync_copy(idx_hbm.at[pl.ds(p0 * KSLOTS, G * KSLOTS)], idxv)
        pltpu.sync_copy(d2_hbm.at[pl.ds(p0 * KSLOTS, G * KSLOTS)], d2v)
        pltpu.async_copy(feats_hbm.at[idxv], rows, sem).wait()
        for px in range(G):
            if px % 2 == 0:
                d2pair = d2v[pl.ds(px * KSLOTS, 2 * KSLOTS)]
                dn = jnp.minimum(jnp.maximum(d2pair * inv_r2, 0.001), 1.0)
                bb = lax.bitcast_convert_type(dn, jnp.int32)
                yy = lax.bitcast_convert_type(
                    lax.shift_right_logical(bb, 1) + jnp.int32(0x1FBD1DF5),
                    jnp.float32)
                yy = 0.5 * (yy + dn / yy)
                yy = 0.5 * (yy + dn / yy)
                avec = 1.0 - yy
            j0 = (px % 2) * KSLOTS
            accs = [jnp.zeros((L,), jnp.float32) for _ in range(4)]
            T = jnp.float32(1.0)
            for k in range(KSLOTS):
                ak = jnp.sum(jnp.where(lanes == j0 + k, avec, 0.0))
                ws = ak * T
                T = T * (1.0 - ak)
                for cb in range(4):
                    r = rows[px * KSLOTS + k, pl.ds(cb * L, L)]
                    accs[cb] = accs[cb] + ws * r
            for cb in range(4):
                outv[px, pl.ds(cb * L, L)] = accs[cb]
        pltpu.sync_copy(outv, out_hbm.at[pl.ds(p0, G)])
        return carry

    lax.fori_loop(0, NCHUNK, chunk, 0, unroll=False)


@jax.jit
def kernel(pts3D, src, default_feature):
    B, Cc, P = src.shape
    H = W = SIZE
    x = -pts3D[..., 0]
    y = -pts3D[..., 1]
    z = pts3D[..., 2]
    fx = (1.0 - x) * W / 2.0 - 0.5
    fy = (1.0 - y) * H / 2.0 - 0.5
    ix0 = jnp.floor(fx).astype(jnp.int32)
    iy0 = jnp.floor(fy).astype(jnp.int32)
    offs = jnp.array([-1, 0, 1, 2], dtype=jnp.int32)
    dyg, dxg = jnp.meshgrid(offs, offs, indexing="ij")
    dy = dyg.reshape(-1)
    dx = dxg.reshape(-1)
    iy = iy0[..., None] + dy[None, None, :]
    ix = ix0[..., None] + dx[None, None, :]
    cx = 1.0 - (2.0 * ix.astype(x.dtype) + 1.0) / W
    cy = 1.0 - (2.0 * iy.astype(y.dtype) + 1.0) / H
    d2 = (x[..., None] - cx) ** 2 + (y[..., None] - cy) ** 2
    inb = (ix >= 0) & (ix < W) & (iy >= 0) & (iy < H)
    valid = inb & (d2 < R2)
    bidx = jnp.arange(B, dtype=jnp.int32)[:, None, None]
    pix = bidx * (H * W) + iy * W + ix
    pix = jnp.where(valid, pix, B * H * W)
    zv = jnp.where(valid, z[..., None], jnp.inf)
    pid = jnp.broadcast_to(
        bidx * P + jnp.arange(P, dtype=jnp.int32)[None, :, None], pix.shape)
    N = B * P * 16
    pix_s, _, pid_s, d2_s = lax.sort(
        (pix.reshape(-1), zv.reshape(-1), pid.reshape(-1), d2.reshape(-1)),
        num_keys=2, is_stable=True)
    q = jnp.arange(B * H * W, dtype=jnp.int32)
    first = jnp.searchsorted(pix_s, q, side="left").astype(jnp.int32)
    last = jnp.searchsorted(pix_s, q, side="right").astype(jnp.int32)
    count = jnp.minimum(last - first, KSLOTS)
    karr = jnp.arange(KSLOTS, dtype=jnp.int32)
    gidx = jnp.where(karr[None, :] < count[:, None], first[:, None] + karr, N)
    pid_p = jnp.concatenate([pid_s, jnp.zeros((8,), jnp.int32)])
    d2_p = jnp.concatenate([d2_s, jnp.full((8,), 1e9, jnp.float32)])
    idx8 = pid_p[gidx].reshape(-1)
    d28 = d2_p[gidx].reshape(-1)
    feats = src.transpose(0, 2, 1).reshape(B * P, Cc)
    feats = jnp.pad(feats, ((0, 0), (0, CP - Cc)))

    compose = functools.partial(
        pl.kernel,
        out_type=jax.ShapeDtypeStruct((NPIX, C), jnp.float32),
        mesh=plsc.VectorSubcoreMesh(core_axis_name="c", subcore_axis_name="s"),
        compiler_params=pltpu.CompilerParams(needs_layout_passes=False),
        scratch_types=[
            pltpu.VMEM((G * KSLOTS,), jnp.int32),
            pltpu.VMEM((G * KSLOTS,), jnp.float32),
            pltpu.VMEM((G * KSLOTS, CP), jnp.float32),
            pltpu.VMEM((G, C), jnp.float32),
            pltpu.SemaphoreType.DMA,
        ],
    )(_compose_kernel)
    out = compose(idx8, d28, feats)
    return out.reshape(B, H, W, C).transpose(0, 3, 1, 2)
